# Initial kernel scaffold; baseline (speedup 1.0000x reference)
import jax
import jax.numpy as jnp
from jax.experimental import pallas as pl


def kernel(x, edge_index, W1, b1, W2, b2):
    raise NotImplementedError("write your pallas kernel here")


# speedup vs baseline: 7.8760x; 7.8760x over previous
import functools

import jax
import jax.numpy as jnp
from jax import lax
from jax.experimental import pallas as pl
from jax.experimental.pallas import tpu as pltpu
from jax.experimental.pallas import tpu_sc as plsc

N = 10000
NC, NS = 2, 16
NT = NC * NS
NPAD = 10240
CHUNK = 128
CPT = 80
EPAD = NT * CPT * CHUNK
RPT = NPAD // NS


def _mesh():
    return plsc.VectorSubcoreMesh(
        core_axis_name="c", subcore_axis_name="s", num_cores=NC, num_subcores=NS
    )


def _deg_partials(dst1d):
    @functools.partial(
        pl.kernel,
        out_type=jax.ShapeDtypeStruct((NC * NPAD,), jnp.float32),
        mesh=_mesh(),
        scratch_types=[
            pltpu.VMEM((CHUNK,), jnp.int32),
            pltpu.VMEM((CHUNK,), jnp.float32),
            pltpu.VMEM((RPT,), jnp.float32),
            pltpu.VMEM_SHARED((NPAD,), jnp.float32),
        ],
    )
    def k(dst_hbm, out_hbm, idx_v, ones_v, z_v, acc_sh):
        c = lax.axis_index("c")
        s = lax.axis_index("s")

        def fill_ones(i, _):
            ones_v[pl.ds(i * 16, 16)] = jnp.ones((16,), jnp.float32)
            return 0

        lax.fori_loop(0, CHUNK // 16, fill_ones, 0)

        def fill_z(i, _):
            z_v[pl.ds(i * 16, 16)] = jnp.zeros((16,), jnp.float32)
            return 0

        lax.fori_loop(0, RPT // 16, fill_z, 0)
        pltpu.sync_copy(z_v, acc_sh.at[pl.ds(s * RPT, RPT)])
        plsc.subcore_barrier()

        base = (c * NS + s) * CPT * CHUNK

        def body(j, _):
            pltpu.sync_copy(dst_hbm.at[pl.ds(base + j * CHUNK, CHUNK)], idx_v)
            pltpu.sync_copy(ones_v, acc_sh.at[idx_v], add=True)
            return 0

        lax.fori_loop(0, CPT, body, 0)
        plsc.subcore_barrier()
        pltpu.sync_copy(
            acc_sh.at[pl.ds(s * RPT, RPT)],
            out_hbm.at[pl.ds(c * NPAD + s * RPT, RPT)],
        )

    return k(dst1d)


def _msg_partials(hp, src1d, dst1d, feat):
    @functools.partial(
        pl.kernel,
        out_type=jax.ShapeDtypeStruct((NC, NPAD, feat), jnp.float32),
        mesh=_mesh(),
        scratch_types=[
            pltpu.VMEM((CHUNK,), jnp.int32),
            pltpu.VMEM((CHUNK,), jnp.int32),
            pltpu.VMEM((CHUNK, feat), jnp.float32),
            pltpu.VMEM_SHARED((NPAD, feat), jnp.float32),
            pltpu.SemaphoreType.DMA,
        ],
    )
    def k(h_hbm, src_hbm, dst_hbm, out_hbm, si_v, di_v, rows_v, acc_sh, sem):
        c = lax.axis_index("c")
        s = lax.axis_index("s")

        def zrow(i, _):
            def zcol(t, _):
                rows_v[i, pl.ds(t * 16, 16)] = jnp.zeros((16,), jnp.float32)
                return 0

            lax.fori_loop(0, feat // 16, zcol, 0)
            return 0

        lax.fori_loop(0, CHUNK, zrow, 0)

        def zacc(q, _):
            pltpu.sync_copy(rows_v, acc_sh.at[pl.ds(s * RPT + q * CHUNK, CHUNK)])
            return 0

        lax.fori_loop(0, RPT // CHUNK, zacc, 0)
        plsc.subcore_barrier()

        base = (c * NS + s) * CPT * CHUNK

        def body(j, _):
            pltpu.sync_copy(src_hbm.at[pl.ds(base + j * CHUNK, CHUNK)], si_v)
            pltpu.sync_copy(dst_hbm.at[pl.ds(base + j * CHUNK, CHUNK)], di_v)
            pltpu.async_copy(h_hbm.at[si_v], rows_v, sem).wait()
            pltpu.sync_copy(rows_v, acc_sh.at[di_v], add=True)
            return 0

        lax.fori_loop(0, CPT, body, 0)
        plsc.subcore_barrier()
        pltpu.sync_copy(
            acc_sh.at[pl.ds(s * RPT, RPT)], out_hbm.at[c, pl.ds(s * RPT, RPT)]
        )

    return k(hp, src1d, dst1d)


def _tc_first(dp, xp, W1):
    def body(dp_ref, x_ref, w_ref, o_ref):
        deg = dp_ref[0, :] + dp_ref[1, :] + 1.0
        dinv = lax.rsqrt(deg)
        h = jnp.dot(x_ref[...], w_ref[...], preferred_element_type=jnp.float32)
        o_ref[...] = h * dinv[:, None]

    return pl.pallas_call(
        body, out_shape=jax.ShapeDtypeStruct((NPAD, 128), jnp.float32)
    )(dp, xp, W1)


def _tc_mid(P, hp, dp, b1, W2):
    def body(p_ref, h_ref, dp_ref, b_ref, w_ref, o_ref):
        deg = dp_ref[0, :] + dp_ref[1, :] + 1.0
        dinv = lax.rsqrt(deg)
        z = (p_ref[0] + p_ref[1] + h_ref[...]) * dinv[:, None] + b_ref[...][None, :]
        z = jnp.maximum(z, 0.0)
        h2 = jnp.dot(z, w_ref[...], preferred_element_type=jnp.float32)
        h2 = h2 * dinv[:, None]
        o_ref[...] = jnp.concatenate([h2, jnp.zeros_like(h2)], axis=1)

    return pl.pallas_call(
        body, out_shape=jax.ShapeDtypeStruct((NPAD, 128), jnp.float32)
    )(P, hp, dp, b1, W2)


def _tc_last(P, hp, dp, b2):
    def body(p_ref, h_ref, dp_ref, b_ref, o_ref):
        deg = dp_ref[0, :] + dp_ref[1, :] + 1.0
        dinv = lax.rsqrt(deg)
        zf = (p_ref[0] + p_ref[1] + h_ref[...]) * dinv[:, None]
        z = zf[:, :64] + b_ref[...][None, :]
        m = jnp.max(z, axis=1, keepdims=True)
        e = jnp.exp(z - m)
        lse = jnp.log(jnp.sum(e, axis=1, keepdims=True))
        o_ref[...] = z - m - lse

    return pl.pallas_call(
        body, out_shape=jax.ShapeDtypeStruct((NPAD, 64), jnp.float32)
    )(P, hp, dp, b2)


@jax.jit
def _run(x, edge_index, W1, b1, W2, b2):
    src = edge_index[0]
    dst = edge_index[1]
    pad = jnp.full((EPAD - src.shape[0],), N, dtype=jnp.int32)
    src1d = jnp.concatenate([src, pad])
    dst1d = jnp.concatenate([dst, pad])
    xp = jnp.zeros((NPAD, x.shape[1]), x.dtype).at[:N].set(x)

    dp = _deg_partials(dst1d).reshape(NC, NPAD)
    hp1 = _tc_first(dp, xp, W1)
    P1 = _msg_partials(hp1, src1d, dst1d, 128)
    hp2 = _tc_mid(P1, hp1, dp, b1, W2)
    P2 = _msg_partials(hp2, src1d, dst1d, 128)
    out = _tc_last(P2, hp2, dp, b2)
    return out[:N]


def kernel(x, edge_index, W1, b1, W2, b2):
    return _run(x, edge_index, W1, b1, W2, b2)


# speedup vs baseline: 9.8036x; 1.2447x over previous
import functools

import jax
import jax.numpy as jnp
from jax import lax
from jax.experimental import pallas as pl
from jax.experimental.pallas import tpu as pltpu
from jax.experimental.pallas import tpu_sc as plsc

N = 10000
NC, NS = 2, 16
NT = NC * NS
NPAD = 10240
CHUNK = 128
CPT = 80
EPAD = NT * CPT * CHUNK
RPT = NPAD // NS


def _mesh():
    return plsc.VectorSubcoreMesh(
        core_axis_name="c", subcore_axis_name="s", num_cores=NC, num_subcores=NS
    )


def _deg_partials(dst1d):
    @functools.partial(
        pl.kernel,
        out_type=jax.ShapeDtypeStruct((NC * NPAD,), jnp.float32),
        mesh=_mesh(),
        scratch_types=[
            pltpu.VMEM((CHUNK,), jnp.int32),
            pltpu.VMEM((CHUNK,), jnp.float32),
            pltpu.VMEM((RPT,), jnp.float32),
            pltpu.VMEM_SHARED((NPAD,), jnp.float32),
        ],
    )
    def k(dst_hbm, out_hbm, idx_v, ones_v, z_v, acc_sh):
        c = lax.axis_index("c")
        s = lax.axis_index("s")

        def fill_ones(i, _):
            ones_v[pl.ds(i * 16, 16)] = jnp.ones((16,), jnp.float32)
            return 0

        lax.fori_loop(0, CHUNK // 16, fill_ones, 0)

        def fill_z(i, _):
            z_v[pl.ds(i * 16, 16)] = jnp.zeros((16,), jnp.float32)
            return 0

        lax.fori_loop(0, RPT // 16, fill_z, 0)
        pltpu.sync_copy(z_v, acc_sh.at[pl.ds(s * RPT, RPT)])
        plsc.subcore_barrier()

        base = (c * NS + s) * CPT * CHUNK

        def body(j, _):
            pltpu.sync_copy(dst_hbm.at[pl.ds(base + j * CHUNK, CHUNK)], idx_v)
            pltpu.sync_copy(ones_v, acc_sh.at[idx_v], add=True)
            return 0

        lax.fori_loop(0, CPT, body, 0)
        plsc.subcore_barrier()
        pltpu.sync_copy(
            acc_sh.at[pl.ds(s * RPT, RPT)],
            out_hbm.at[pl.ds(c * NPAD + s * RPT, RPT)],
        )

    return k(dst1d)


def _msg_partials(hp, src1d, dst1d, feat):
    @functools.partial(
        pl.kernel,
        out_type=jax.ShapeDtypeStruct((NC, NPAD, feat), jnp.float32),
        mesh=_mesh(),
        scratch_types=[
            pltpu.VMEM((CPT * CHUNK,), jnp.int32),
            pltpu.VMEM((CHUNK,), jnp.int32),
            pltpu.VMEM((CHUNK,), jnp.int32),
            pltpu.VMEM((CHUNK, feat), jnp.float32),
            pltpu.VMEM((CHUNK, feat), jnp.float32),
            pltpu.VMEM_SHARED((NPAD, feat), jnp.float32),
            pltpu.SemaphoreType.DMA,
            pltpu.SemaphoreType.DMA,
            pltpu.SemaphoreType.DMA,
            pltpu.SemaphoreType.DMA,
        ],
    )
    def k(h_hbm, src_hbm, dst_hbm, out_hbm, si_all, di_a, di_b, rows_a,
          rows_b, acc_sh, sga, sgb, sda, sdb):
        c = lax.axis_index("c")
        s = lax.axis_index("s")

        def zrow(i, _):
            def zcol(t, _):
                rows_a[i, pl.ds(t * 16, 16)] = jnp.zeros((16,), jnp.float32)
                return 0

            lax.fori_loop(0, feat // 16, zcol, 0)
            return 0

        lax.fori_loop(0, CHUNK, zrow, 0)

        def zacc(q, _):
            pltpu.sync_copy(rows_a, acc_sh.at[pl.ds(s * RPT + q * CHUNK, CHUNK)])
            return 0

        lax.fori_loop(0, RPT // CHUNK, zacc, 0)
        plsc.subcore_barrier()

        base = (c * NS + s) * CPT * CHUNK
        pltpu.sync_copy(src_hbm.at[pl.ds(base, CPT * CHUNK)], si_all)

        def start(j, rows_v, di_v, sg, sd):
            pltpu.async_copy(dst_hbm.at[pl.ds(base + j * CHUNK, CHUNK)], di_v, sd)
            pltpu.async_copy(
                h_hbm.at[si_all.at[pl.ds(j * CHUNK, CHUNK)]], rows_v, sg
            )

        def finish(j, rows_v, di_v, sg, sd):
            pltpu.make_async_copy(
                dst_hbm.at[pl.ds(base + j * CHUNK, CHUNK)], di_v, sd
            ).wait()
            pltpu.make_async_copy(
                h_hbm.at[si_all.at[pl.ds(j * CHUNK, CHUNK)]], rows_v, sg
            ).wait()
            pltpu.sync_copy(rows_v, acc_sh.at[di_v], add=True)

        start(0, rows_a, di_a, sga, sda)

        def body(t, _):
            j0 = 2 * t
            start(j0 + 1, rows_b, di_b, sgb, sdb)
            finish(j0, rows_a, di_a, sga, sda)

            @pl.when(t + 1 < CPT // 2)
            def _():
                start(j0 + 2, rows_a, di_a, sga, sda)

            finish(j0 + 1, rows_b, di_b, sgb, sdb)
            return 0

        lax.fori_loop(0, CPT // 2, body, 0)
        plsc.subcore_barrier()
        pltpu.sync_copy(
            acc_sh.at[pl.ds(s * RPT, RPT)], out_hbm.at[c, pl.ds(s * RPT, RPT)]
        )

    return k(hp, src1d, dst1d)


def _tc_first(dp, xp, W1):
    def body(dp_ref, x_ref, w_ref, o_ref):
        deg = dp_ref[0, :] + dp_ref[1, :] + 1.0
        dinv = lax.rsqrt(deg)
        h = jnp.dot(x_ref[...], w_ref[...], preferred_element_type=jnp.float32)
        o_ref[...] = h * dinv[:, None]

    return pl.pallas_call(
        body, out_shape=jax.ShapeDtypeStruct((NPAD, 128), jnp.float32)
    )(dp, xp, W1)


def _tc_mid(P, hp, dp, b1, W2):
    def body(p_ref, h_ref, dp_ref, b_ref, w_ref, o_ref):
        deg = dp_ref[0, :] + dp_ref[1, :] + 1.0
        dinv = lax.rsqrt(deg)
        z = (p_ref[0] + p_ref[1] + h_ref[...]) * dinv[:, None] + b_ref[...][None, :]
        z = jnp.maximum(z, 0.0)
        h2 = jnp.dot(z, w_ref[...], preferred_element_type=jnp.float32)
        h2 = h2 * dinv[:, None]
        o_ref[...] = jnp.concatenate([h2, jnp.zeros_like(h2)], axis=1)

    return pl.pallas_call(
        body, out_shape=jax.ShapeDtypeStruct((NPAD, 128), jnp.float32)
    )(P, hp, dp, b1, W2)


def _tc_last(P, hp, dp, b2):
    def body(p_ref, h_ref, dp_ref, b_ref, o_ref):
        deg = dp_ref[0, :] + dp_ref[1, :] + 1.0
        dinv = lax.rsqrt(deg)
        zf = (p_ref[0] + p_ref[1] + h_ref[...]) * dinv[:, None]
        z = zf[:, :64] + b_ref[...][None, :]
        m = jnp.max(z, axis=1, keepdims=True)
        e = jnp.exp(z - m)
        lse = jnp.log(jnp.sum(e, axis=1, keepdims=True))
        o_ref[...] = z - m - lse

    return pl.pallas_call(
        body, out_shape=jax.ShapeDtypeStruct((NPAD, 64), jnp.float32)
    )(P, hp, dp, b2)


@jax.jit
def _run(x, edge_index, W1, b1, W2, b2):
    src = edge_index[0]
    dst = edge_index[1]
    pad = jnp.full((EPAD - src.shape[0],), N, dtype=jnp.int32)
    src1d = jnp.concatenate([src, pad])
    dst1d = jnp.concatenate([dst, pad])
    xp = jnp.zeros((NPAD, x.shape[1]), x.dtype).at[:N].set(x)

    dp = _deg_partials(dst1d).reshape(NC, NPAD)
    hp1 = _tc_first(dp, xp, W1)
    P1 = _msg_partials(hp1, src1d, dst1d, 128)
    hp2 = _tc_mid(P1, hp1, dp, b1, W2)
    P2 = _msg_partials(hp2, src1d, dst1d, 128)
    out = _tc_last(P2, hp2, dp, b2)
    return out[:N]


def kernel(x, edge_index, W1, b1, W2, b2):
    return _run(x, edge_index, W1, b1, W2, b2)


# speedup vs baseline: 31.4829x; 3.2114x over previous
import functools

import jax
import jax.numpy as jnp
from jax import lax
from jax.experimental import pallas as pl
from jax.experimental.pallas import tpu as pltpu
from jax.experimental.pallas import tpu_sc as plsc

N = 10000
NC, NS = 2, 16
NT = NC * NS
NPAD = 10240
CHUNK = 128
CPT = 80
EPAD = NT * CPT * CHUNK
RPT = NPAD // NS


def _mesh():
    return plsc.VectorSubcoreMesh(
        core_axis_name="c", subcore_axis_name="s", num_cores=NC, num_subcores=NS
    )


def _deg_partials(dst1d):
    @functools.partial(
        pl.kernel,
        out_type=jax.ShapeDtypeStruct((NC * NPAD,), jnp.float32),
        mesh=_mesh(),
        scratch_types=[
            pltpu.VMEM((CHUNK,), jnp.int32),
            pltpu.VMEM((CHUNK,), jnp.float32),
            pltpu.VMEM((RPT,), jnp.float32),
            pltpu.VMEM_SHARED((NPAD,), jnp.float32),
        ],
    )
    def k(dst_hbm, out_hbm, idx_v, ones_v, z_v, acc_sh):
        c = lax.axis_index("c")
        s = lax.axis_index("s")

        def fill_ones(i, _):
            ones_v[pl.ds(i * 16, 16)] = jnp.ones((16,), jnp.float32)
            return 0

        lax.fori_loop(0, CHUNK // 16, fill_ones, 0)

        def fill_z(i, _):
            z_v[pl.ds(i * 16, 16)] = jnp.zeros((16,), jnp.float32)
            return 0

        lax.fori_loop(0, RPT // 16, fill_z, 0)
        pltpu.sync_copy(z_v, acc_sh.at[pl.ds(s * RPT, RPT)])
        plsc.subcore_barrier()

        base = (c * NS + s) * CPT * CHUNK

        def body(j, _):
            pltpu.sync_copy(dst_hbm.at[pl.ds(base + j * CHUNK, CHUNK)], idx_v)
            pltpu.sync_copy(ones_v, acc_sh.at[idx_v], add=True)
            return 0

        lax.fori_loop(0, CPT, body, 0)
        plsc.subcore_barrier()
        pltpu.sync_copy(
            acc_sh.at[pl.ds(s * RPT, RPT)],
            out_hbm.at[pl.ds(c * NPAD + s * RPT, RPT)],
        )

    return k(dst1d)


def _msg_partials(hp, src1d, dst1d, feat):
    @functools.partial(
        pl.kernel,
        out_type=jax.ShapeDtypeStruct((NC, NPAD, feat), jnp.float32),
        mesh=_mesh(),
        scratch_types=[
            pltpu.VMEM((CPT * CHUNK,), jnp.int32),
            pltpu.VMEM((CHUNK,), jnp.int32),
            pltpu.VMEM((CHUNK,), jnp.int32),
            pltpu.VMEM((CHUNK, feat), jnp.float32),
            pltpu.VMEM((CHUNK, feat), jnp.float32),
            pltpu.VMEM_SHARED((NPAD, feat), jnp.float32),
            pltpu.SemaphoreType.DMA,
            pltpu.SemaphoreType.DMA,
            pltpu.SemaphoreType.DMA,
            pltpu.SemaphoreType.DMA,
        ],
    )
    def k(h_hbm, src_hbm, dst_hbm, out_hbm, si_all, di_a, di_b, rows_a,
          rows_b, acc_sh, sga, sgb, sda, sdb):
        c = lax.axis_index("c")
        s = lax.axis_index("s")

        def zrow(i, _):
            def zcol(t, _):
                rows_a[i, pl.ds(t * 16, 16)] = jnp.zeros((16,), jnp.float32)
                return 0

            lax.fori_loop(0, feat // 16, zcol, 0)
            return 0

        lax.fori_loop(0, CHUNK, zrow, 0)

        def zacc(q, _):
            pltpu.sync_copy(rows_a, acc_sh.at[pl.ds(s * RPT + q * CHUNK, CHUNK)])
            return 0

        lax.fori_loop(0, RPT // CHUNK, zacc, 0)
        plsc.subcore_barrier()

        base = (c * NS + s) * CPT * CHUNK
        pltpu.sync_copy(src_hbm.at[pl.ds(base, CPT * CHUNK)], si_all)

        def start(j, rows_v, di_v, sg, sd):
            pltpu.async_copy(dst_hbm.at[pl.ds(base + j * CHUNK, CHUNK)], di_v, sd)
            pltpu.async_copy(
                h_hbm.at[si_all.at[pl.ds(j * CHUNK, CHUNK)]], rows_v, sg
            )

        def finish(j, rows_v, di_v, sg, sd):
            pltpu.make_async_copy(
                dst_hbm.at[pl.ds(base + j * CHUNK, CHUNK)], di_v, sd
            ).wait()
            pltpu.make_async_copy(
                h_hbm.at[si_all.at[pl.ds(j * CHUNK, CHUNK)]], rows_v, sg
            ).wait()
            pltpu.sync_copy(rows_v, acc_sh.at[di_v], add=True)

        start(0, rows_a, di_a, sga, sda)

        def body(t, _):
            j0 = 2 * t
            start(j0 + 1, rows_b, di_b, sgb, sdb)
            finish(j0, rows_a, di_a, sga, sda)

            @pl.when(t + 1 < CPT // 2)
            def _():
                start(j0 + 2, rows_a, di_a, sga, sda)

            finish(j0 + 1, rows_b, di_b, sgb, sdb)
            return 0

        lax.fori_loop(0, CPT // 2, body, 0)
        plsc.subcore_barrier()
        pltpu.sync_copy(
            acc_sh.at[pl.ds(s * RPT, RPT)], out_hbm.at[c, pl.ds(s * RPT, RPT)]
        )

    return k(hp, src1d, dst1d)


def _tc_first(dp, xp, W1):
    def body(dp_ref, x_ref, w_ref, o_ref):
        deg = dp_ref[0, :] + dp_ref[1, :] + 1.0
        dinv = lax.rsqrt(deg)
        h = jnp.dot(x_ref[...], w_ref[...], preferred_element_type=jnp.float32)
        o_ref[...] = h * dinv[:, None]

    return pl.pallas_call(
        body, out_shape=jax.ShapeDtypeStruct((NPAD, 128), jnp.float32)
    )(dp, xp, W1)


def _tc_mid(P, hp, dp, b1, W2):
    def body(p_ref, h_ref, dp_ref, b_ref, w_ref, o_ref):
        deg = dp_ref[0, :] + dp_ref[1, :] + 1.0
        dinv = lax.rsqrt(deg)
        z = (p_ref[0] + p_ref[1] + h_ref[...]) * dinv[:, None] + b_ref[...][None, :]
        z = jnp.maximum(z, 0.0)
        h2 = jnp.dot(z, w_ref[...], preferred_element_type=jnp.float32)
        h2 = h2 * dinv[:, None]
        o_ref[...] = jnp.concatenate([h2, jnp.zeros_like(h2)], axis=1)

    return pl.pallas_call(
        body, out_shape=jax.ShapeDtypeStruct((NPAD, 128), jnp.float32)
    )(P, hp, dp, b1, W2)


def _tc_last(P, hp, dp, b2):
    def body(p_ref, h_ref, dp_ref, b_ref, o_ref):
        deg = dp_ref[0, :] + dp_ref[1, :] + 1.0
        dinv = lax.rsqrt(deg)
        zf = (p_ref[0] + p_ref[1] + h_ref[...]) * dinv[:, None]
        z = zf[:, :64] + b_ref[...][None, :]
        m = jnp.max(z, axis=1, keepdims=True)
        e = jnp.exp(z - m)
        lse = jnp.log(jnp.sum(e, axis=1, keepdims=True))
        o_ref[...] = z - m - lse

    return pl.pallas_call(
        body, out_shape=jax.ShapeDtypeStruct((NPAD, 64), jnp.float32)
    )(P, hp, dp, b2)


@jax.jit
def _run(x, edge_index, W1, b1, W2, b2):
    src = edge_index[0]
    dst = edge_index[1]
    pad = N + (jnp.arange(EPAD - src.shape[0], dtype=jnp.int32) % (NPAD - N))
    src1d = jnp.concatenate([src, pad])
    dst1d = jnp.concatenate([dst, pad])
    xp = jnp.zeros((NPAD, x.shape[1]), x.dtype).at[:N].set(x)

    dp = _deg_partials(dst1d).reshape(NC, NPAD)
    hp1 = _tc_first(dp, xp, W1)
    P1 = _msg_partials(hp1, src1d, dst1d, 128)
    hp2 = _tc_mid(P1, hp1, dp, b1, W2)
    P2 = _msg_partials(hp2, src1d, dst1d, 128)
    out = _tc_last(P2, hp2, dp, b2)
    return out[:N]


def kernel(x, edge_index, W1, b1, W2, b2):
    return _run(x, edge_index, W1, b1, W2, b2)


# speedup vs baseline: 36.6590x; 1.1644x over previous
import functools

import jax
import jax.numpy as jnp
from jax import lax
from jax.experimental import pallas as pl
from jax.experimental.pallas import tpu as pltpu
from jax.experimental.pallas import tpu_sc as plsc

N = 10000
NC, NS = 2, 16
NT = NC * NS
NPAD = 10240
CHUNK = 128
CPT = 80
EPAD = NT * CPT * CHUNK
RPT = NPAD // NS


def _mesh():
    return plsc.VectorSubcoreMesh(
        core_axis_name="c", subcore_axis_name="s", num_cores=NC, num_subcores=NS
    )


def _deg_partials(dst1d):
    @functools.partial(
        pl.kernel,
        out_type=jax.ShapeDtypeStruct((NC * NPAD,), jnp.float32),
        mesh=_mesh(),
        scratch_types=[
            pltpu.VMEM((CHUNK,), jnp.int32),
            pltpu.VMEM((CHUNK,), jnp.int32),
            pltpu.VMEM((CHUNK,), jnp.float32),
            pltpu.VMEM((RPT,), jnp.float32),
            pltpu.VMEM_SHARED((NPAD,), jnp.float32),
            pltpu.SemaphoreType.DMA,
            pltpu.SemaphoreType.DMA,
        ],
    )
    def k(dst_hbm, out_hbm, idx_a, idx_b, ones_v, z_v, acc_sh, sa, sb):
        c = lax.axis_index("c")
        s = lax.axis_index("s")

        def fill_ones(i, _):
            ones_v[pl.ds(i * 16, 16)] = jnp.ones((16,), jnp.float32)
            return 0

        lax.fori_loop(0, CHUNK // 16, fill_ones, 0)

        def fill_z(i, _):
            z_v[pl.ds(i * 16, 16)] = jnp.zeros((16,), jnp.float32)
            return 0

        lax.fori_loop(0, RPT // 16, fill_z, 0)
        pltpu.sync_copy(z_v, acc_sh.at[pl.ds(s * RPT, RPT)])
        plsc.subcore_barrier()

        base = (c * NS + s) * CPT * CHUNK

        def start(j, idx_v, sem):
            pltpu.async_copy(dst_hbm.at[pl.ds(base + j * CHUNK, CHUNK)], idx_v, sem)

        def finish(j, idx_v, sem):
            pltpu.make_async_copy(
                dst_hbm.at[pl.ds(base + j * CHUNK, CHUNK)], idx_v, sem
            ).wait()
            pltpu.sync_copy(ones_v, acc_sh.at[idx_v], add=True)

        start(0, idx_a, sa)

        def body(t, _):
            j0 = 2 * t
            start(j0 + 1, idx_b, sb)
            finish(j0, idx_a, sa)

            @pl.when(t + 1 < CPT // 2)
            def _():
                start(j0 + 2, idx_a, sa)

            finish(j0 + 1, idx_b, sb)
            return 0

        lax.fori_loop(0, CPT // 2, body, 0)
        plsc.subcore_barrier()
        pltpu.sync_copy(
            acc_sh.at[pl.ds(s * RPT, RPT)],
            out_hbm.at[pl.ds(c * NPAD + s * RPT, RPT)],
        )

    return k(dst1d)


def _msg_partials(hp, src1d, dst1d, feat):
    params = (
        pltpu.CompilerParams(use_tc_tiling_on_sc=False) if feat != 128 else None
    )

    @functools.partial(
        pl.kernel,
        out_type=jax.ShapeDtypeStruct((NC, NPAD, feat), jnp.float32),
        mesh=_mesh(),
        compiler_params=params,
        scratch_types=[
            pltpu.VMEM((CPT * CHUNK,), jnp.int32),
            pltpu.VMEM((CHUNK,), jnp.int32),
            pltpu.VMEM((CHUNK,), jnp.int32),
            pltpu.VMEM((CHUNK, feat), jnp.float32),
            pltpu.VMEM((CHUNK, feat), jnp.float32),
            pltpu.VMEM_SHARED((NPAD, feat), jnp.float32),
            pltpu.SemaphoreType.DMA,
            pltpu.SemaphoreType.DMA,
            pltpu.SemaphoreType.DMA,
            pltpu.SemaphoreType.DMA,
        ],
    )
    def k(h_hbm, src_hbm, dst_hbm, out_hbm, si_all, di_a, di_b, rows_a,
          rows_b, acc_sh, sga, sgb, sda, sdb):
        c = lax.axis_index("c")
        s = lax.axis_index("s")

        def zrow(i, _):
            def zcol(t, _):
                rows_a[i, pl.ds(t * 16, 16)] = jnp.zeros((16,), jnp.float32)
                return 0

            lax.fori_loop(0, feat // 16, zcol, 0)
            return 0

        lax.fori_loop(0, CHUNK, zrow, 0)

        def zacc(q, _):
            pltpu.sync_copy(rows_a, acc_sh.at[pl.ds(s * RPT + q * CHUNK, CHUNK)])
            return 0

        lax.fori_loop(0, RPT // CHUNK, zacc, 0)
        plsc.subcore_barrier()

        base = (c * NS + s) * CPT * CHUNK
        pltpu.sync_copy(src_hbm.at[pl.ds(base, CPT * CHUNK)], si_all)

        def start(j, rows_v, di_v, sg, sd):
            pltpu.async_copy(dst_hbm.at[pl.ds(base + j * CHUNK, CHUNK)], di_v, sd)
            pltpu.async_copy(
                h_hbm.at[si_all.at[pl.ds(j * CHUNK, CHUNK)]], rows_v, sg
            )

        def finish(j, rows_v, di_v, sg, sd):
            pltpu.make_async_copy(
                dst_hbm.at[pl.ds(base + j * CHUNK, CHUNK)], di_v, sd
            ).wait()
            pltpu.make_async_copy(
                h_hbm.at[si_all.at[pl.ds(j * CHUNK, CHUNK)]], rows_v, sg
            ).wait()
            pltpu.sync_copy(rows_v, acc_sh.at[di_v], add=True)

        start(0, rows_a, di_a, sga, sda)

        def body(t, _):
            j0 = 2 * t
            start(j0 + 1, rows_b, di_b, sgb, sdb)
            finish(j0, rows_a, di_a, sga, sda)

            @pl.when(t + 1 < CPT // 2)
            def _():
                start(j0 + 2, rows_a, di_a, sga, sda)

            finish(j0 + 1, rows_b, di_b, sgb, sdb)
            return 0

        lax.fori_loop(0, CPT // 2, body, 0)
        plsc.subcore_barrier()
        pltpu.sync_copy(
            acc_sh.at[pl.ds(s * RPT, RPT)], out_hbm.at[c, pl.ds(s * RPT, RPT)]
        )

    return k(hp, src1d, dst1d)


def _tc_matmul(xp, W1):
    def body(x_ref, w_ref, o_ref):
        o_ref[...] = jnp.dot(
            x_ref[...], w_ref[...], preferred_element_type=jnp.float32
        )

    return pl.pallas_call(
        body, out_shape=jax.ShapeDtypeStruct((NPAD, 128), jnp.float32)
    )(xp, W1)


def _tc_scale(dp, h):
    def body(dp_ref, h_ref, o_ref):
        deg = dp_ref[0, :] + dp_ref[1, :] + 1.0
        dinv = lax.rsqrt(deg)
        o_ref[...] = h_ref[...] * dinv[:, None]

    return pl.pallas_call(
        body, out_shape=jax.ShapeDtypeStruct((NPAD, 128), jnp.float32)
    )(dp, h)


def _tc_mid(P, hp, dp, b1, W2):
    def body(p_ref, h_ref, dp_ref, b_ref, w_ref, o_ref):
        deg = dp_ref[0, :] + dp_ref[1, :] + 1.0
        dinv = lax.rsqrt(deg)
        z = (p_ref[0] + p_ref[1] + h_ref[...]) * dinv[:, None] + b_ref[...][None, :]
        z = jnp.maximum(z, 0.0)
        h2 = jnp.dot(z, w_ref[...], preferred_element_type=jnp.float32)
        o_ref[...] = h2 * dinv[:, None]

    return pl.pallas_call(
        body, out_shape=jax.ShapeDtypeStruct((NPAD, 64), jnp.float32)
    )(P, hp, dp, b1, W2)


def _tc_last(P, hp, dp, b2):
    def body(p_ref, h_ref, dp_ref, b_ref, o_ref):
        deg = dp_ref[0, :] + dp_ref[1, :] + 1.0
        dinv = lax.rsqrt(deg)
        z = (p_ref[0] + p_ref[1] + h_ref[...]) * dinv[:, None] + b_ref[...][None, :]
        m = jnp.max(z, axis=1, keepdims=True)
        e = jnp.exp(z - m)
        lse = jnp.log(jnp.sum(e, axis=1, keepdims=True))
        o_ref[...] = z - m - lse

    return pl.pallas_call(
        body, out_shape=jax.ShapeDtypeStruct((NPAD, 64), jnp.float32)
    )(P, hp, dp, b2)


@jax.jit
def _run(x, edge_index, W1, b1, W2, b2):
    src = edge_index[0]
    dst = edge_index[1]
    pad = N + (jnp.arange(EPAD - src.shape[0], dtype=jnp.int32) % (NPAD - N))
    src1d = jnp.concatenate([src, pad])
    dst1d = jnp.concatenate([dst, pad])
    xp = jnp.zeros((NPAD, x.shape[1]), x.dtype).at[:N].set(x)

    h1 = _tc_matmul(xp, W1)
    dp = _deg_partials(dst1d).reshape(NC, NPAD)
    hp1 = _tc_scale(dp, h1)
    P1 = _msg_partials(hp1, src1d, dst1d, 128)
    hp2 = _tc_mid(P1, hp1, dp, b1, W2)
    P2 = _msg_partials(hp2, src1d, dst1d, 64)
    out = _tc_last(P2, hp2, dp, b2)
    return out[:N]


def kernel(x, edge_index, W1, b1, W2, b2):
    return _run(x, edge_index, W1, b1, W2, b2)


# speedup vs baseline: 36.8182x; 1.0043x over previous
import functools

import jax
import jax.numpy as jnp
from jax import lax
from jax.experimental import pallas as pl
from jax.experimental.pallas import tpu as pltpu
from jax.experimental.pallas import tpu_sc as plsc

N = 10000
NC, NS = 2, 16
NT = NC * NS
NPAD = 10240
CHUNK = 128
CPT = 80
EPAD = NT * CPT * CHUNK
RPT = NPAD // NS


def _mesh():
    return plsc.VectorSubcoreMesh(
        core_axis_name="c", subcore_axis_name="s", num_cores=NC, num_subcores=NS
    )


def _deg_partials(dst1d):
    @functools.partial(
        pl.kernel,
        out_type=jax.ShapeDtypeStruct((NC * NPAD,), jnp.float32),
        mesh=_mesh(),
        scratch_types=[
            pltpu.VMEM((CHUNK,), jnp.int32),
            pltpu.VMEM((CHUNK,), jnp.int32),
            pltpu.VMEM((CHUNK,), jnp.float32),
            pltpu.VMEM((RPT,), jnp.float32),
            pltpu.VMEM_SHARED((NPAD,), jnp.float32),
            pltpu.SemaphoreType.DMA,
            pltpu.SemaphoreType.DMA,
        ],
    )
    def k(dst_hbm, out_hbm, idx_a, idx_b, ones_v, z_v, acc_sh, sa, sb):
        c = lax.axis_index("c")
        s = lax.axis_index("s")

        def fill_ones(i, _):
            ones_v[pl.ds(i * 16, 16)] = jnp.ones((16,), jnp.float32)
            return 0

        lax.fori_loop(0, CHUNK // 16, fill_ones, 0)

        def fill_z(i, _):
            z_v[pl.ds(i * 16, 16)] = jnp.zeros((16,), jnp.float32)
            return 0

        lax.fori_loop(0, RPT // 16, fill_z, 0)
        pltpu.sync_copy(z_v, acc_sh.at[pl.ds(s * RPT, RPT)])
        plsc.subcore_barrier()

        base = (c * NS + s) * CPT * CHUNK

        def start(j, idx_v, sem):
            pltpu.async_copy(dst_hbm.at[pl.ds(base + j * CHUNK, CHUNK)], idx_v, sem)

        def finish(j, idx_v, sem):
            pltpu.make_async_copy(
                dst_hbm.at[pl.ds(base + j * CHUNK, CHUNK)], idx_v, sem
            ).wait()
            pltpu.sync_copy(ones_v, acc_sh.at[idx_v], add=True)

        start(0, idx_a, sa)

        def body(t, _):
            j0 = 2 * t
            start(j0 + 1, idx_b, sb)
            finish(j0, idx_a, sa)

            @pl.when(t + 1 < CPT // 2)
            def _():
                start(j0 + 2, idx_a, sa)

            finish(j0 + 1, idx_b, sb)
            return 0

        lax.fori_loop(0, CPT // 2, body, 0)
        plsc.subcore_barrier()
        pltpu.sync_copy(
            acc_sh.at[pl.ds(s * RPT, RPT)],
            out_hbm.at[pl.ds(c * NPAD + s * RPT, RPT)],
        )

    return k(dst1d)


def _msg_partials(hp, src1d, dst1d, feat):
    params = (
        pltpu.CompilerParams(use_tc_tiling_on_sc=False) if feat != 128 else None
    )

    @functools.partial(
        pl.kernel,
        out_type=jax.ShapeDtypeStruct((NC, NPAD, feat), jnp.float32),
        mesh=_mesh(),
        compiler_params=params,
        scratch_types=[
            pltpu.VMEM((CPT * CHUNK,), jnp.int32),
            pltpu.VMEM((CHUNK,), jnp.int32),
            pltpu.VMEM((CHUNK,), jnp.int32),
            pltpu.VMEM((CHUNK, feat), jnp.float32),
            pltpu.VMEM((CHUNK, feat), jnp.float32),
            pltpu.VMEM_SHARED((NPAD, feat), jnp.float32),
            pltpu.SemaphoreType.DMA,
            pltpu.SemaphoreType.DMA,
            pltpu.SemaphoreType.DMA,
            pltpu.SemaphoreType.DMA,
        ],
    )
    def k(h_hbm, src_hbm, dst_hbm, out_hbm, si_all, di_a, di_b, rows_a,
          rows_b, acc_sh, sga, sgb, sda, sdb):
        c = lax.axis_index("c")
        s = lax.axis_index("s")

        def zrow(i, _):
            def zcol(t, _):
                rows_a[i, pl.ds(t * 16, 16)] = jnp.zeros((16,), jnp.float32)
                return 0

            lax.fori_loop(0, feat // 16, zcol, 0)
            return 0

        lax.fori_loop(0, CHUNK, zrow, 0)

        def zacc(q, _):
            pltpu.sync_copy(rows_a, acc_sh.at[pl.ds(s * RPT + q * CHUNK, CHUNK)])
            return 0

        lax.fori_loop(0, RPT // CHUNK, zacc, 0)
        plsc.subcore_barrier()

        base = (c * NS + s) * CPT * CHUNK
        pltpu.sync_copy(src_hbm.at[pl.ds(base, CPT * CHUNK)], si_all)

        def start(j, rows_v, di_v, sg, sd):
            pltpu.async_copy(dst_hbm.at[pl.ds(base + j * CHUNK, CHUNK)], di_v, sd)
            pltpu.async_copy(
                h_hbm.at[si_all.at[pl.ds(j * CHUNK, CHUNK)]], rows_v, sg
            )

        def finish(j, rows_v, di_v, sg, sd):
            pltpu.make_async_copy(
                dst_hbm.at[pl.ds(base + j * CHUNK, CHUNK)], di_v, sd
            ).wait()
            pltpu.make_async_copy(
                h_hbm.at[si_all.at[pl.ds(j * CHUNK, CHUNK)]], rows_v, sg
            ).wait()
            pltpu.sync_copy(rows_v, acc_sh.at[di_v], add=True)

        start(0, rows_a, di_a, sga, sda)

        def body(t, _):
            j0 = 2 * t
            start(j0 + 1, rows_b, di_b, sgb, sdb)
            finish(j0, rows_a, di_a, sga, sda)

            @pl.when(t + 1 < CPT // 2)
            def _():
                start(j0 + 2, rows_a, di_a, sga, sda)

            finish(j0 + 1, rows_b, di_b, sgb, sdb)
            return 0

        lax.fori_loop(0, CPT // 2, body, 0)
        plsc.subcore_barrier()
        pltpu.sync_copy(
            acc_sh.at[pl.ds(s * RPT, RPT)], out_hbm.at[c, pl.ds(s * RPT, RPT)]
        )

    return k(hp, src1d, dst1d)


def _tc_first(dp, xp, W1):
    def body(dp_ref, x_ref, w_ref, o_ref):
        deg = dp_ref[0, :] + dp_ref[1, :] + 1.0
        dinv = lax.rsqrt(deg)
        h = jnp.dot(x_ref[...], w_ref[...], preferred_element_type=jnp.float32)
        o_ref[...] = h * dinv[:, None]

    return pl.pallas_call(
        body, out_shape=jax.ShapeDtypeStruct((NPAD, 128), jnp.float32)
    )(dp, xp, W1)


def _tc_mid(P, hp, dp, b1, W2):
    def body(p_ref, h_ref, dp_ref, b_ref, w_ref, o_ref):
        deg = dp_ref[0, :] + dp_ref[1, :] + 1.0
        dinv = lax.rsqrt(deg)
        z = (p_ref[0] + p_ref[1] + h_ref[...]) * dinv[:, None] + b_ref[...][None, :]
        z = jnp.maximum(z, 0.0)
        h2 = jnp.dot(z, w_ref[...], preferred_element_type=jnp.float32)
        o_ref[...] = h2 * dinv[:, None]

    return pl.pallas_call(
        body, out_shape=jax.ShapeDtypeStruct((NPAD, 64), jnp.float32)
    )(P, hp, dp, b1, W2)


def _tc_last(P, hp, dp, b2):
    def body(p_ref, h_ref, dp_ref, b_ref, o_ref):
        deg = dp_ref[0, :] + dp_ref[1, :] + 1.0
        dinv = lax.rsqrt(deg)
        z = (p_ref[0] + p_ref[1] + h_ref[...]) * dinv[:, None] + b_ref[...][None, :]
        m = jnp.max(z, axis=1, keepdims=True)
        e = jnp.exp(z - m)
        lse = jnp.log(jnp.sum(e, axis=1, keepdims=True))
        o_ref[...] = z - m - lse

    return pl.pallas_call(
        body, out_shape=jax.ShapeDtypeStruct((NPAD, 64), jnp.float32)
    )(P, hp, dp, b2)


@jax.jit
def _run(x, edge_index, W1, b1, W2, b2):
    src = edge_index[0]
    dst = edge_index[1]
    pad = N + (jnp.arange(EPAD - src.shape[0], dtype=jnp.int32) % (NPAD - N))
    src1d = jnp.concatenate([src, pad])
    dst1d = jnp.concatenate([dst, pad])
    xp = jnp.zeros((NPAD, x.shape[1]), x.dtype).at[:N].set(x)

    dp = _deg_partials(dst1d).reshape(NC, NPAD)
    hp1 = _tc_first(dp, xp, W1)
    P1 = _msg_partials(hp1, src1d, dst1d, 128)
    hp2 = _tc_mid(P1, hp1, dp, b1, W2)
    P2 = _msg_partials(hp2, src1d, dst1d, 64)
    out = _tc_last(P2, hp2, dp, b2)
    return out[:N]


def kernel(x, edge_index, W1, b1, W2, b2):
    return _run(x, edge_index, W1, b1, W2, b2)


# speedup vs baseline: 39.0312x; 1.0601x over previous
import functools

import jax
import jax.numpy as jnp
from jax import lax
from jax.experimental import pallas as pl
from jax.experimental.pallas import tpu as pltpu
from jax.experimental.pallas import tpu_sc as plsc

N = 10000
NC, NS = 2, 16
NT = NC * NS
NPAD = 10240
CHUNK = 128
CPT = 80
EPAD = NT * CPT * CHUNK
RPT = NPAD // NS


def _mesh():
    return plsc.VectorSubcoreMesh(
        core_axis_name="c", subcore_axis_name="s", num_cores=NC, num_subcores=NS
    )


def _deg_partials(dst1d):
    @functools.partial(
        pl.kernel,
        out_type=jax.ShapeDtypeStruct((NC * NPAD,), jnp.float32),
        mesh=_mesh(),
        scratch_types=[
            pltpu.VMEM((CHUNK,), jnp.int32),
            pltpu.VMEM((CHUNK,), jnp.int32),
            pltpu.VMEM((CHUNK,), jnp.float32),
            pltpu.VMEM((RPT,), jnp.float32),
            pltpu.VMEM_SHARED((NPAD,), jnp.float32),
            pltpu.SemaphoreType.DMA,
            pltpu.SemaphoreType.DMA,
        ],
    )
    def k(dst_hbm, out_hbm, idx_a, idx_b, ones_v, z_v, acc_sh, sa, sb):
        c = lax.axis_index("c")
        s = lax.axis_index("s")

        def fill_ones(i, _):
            ones_v[pl.ds(i * 16, 16)] = jnp.ones((16,), jnp.float32)
            return 0

        lax.fori_loop(0, CHUNK // 16, fill_ones, 0)

        def fill_z(i, _):
            z_v[pl.ds(i * 16, 16)] = jnp.zeros((16,), jnp.float32)
            return 0

        lax.fori_loop(0, RPT // 16, fill_z, 0)
        pltpu.sync_copy(z_v, acc_sh.at[pl.ds(s * RPT, RPT)])
        plsc.subcore_barrier()

        base = (c * NS + s) * CPT * CHUNK

        def start(j, idx_v, sem):
            pltpu.async_copy(dst_hbm.at[pl.ds(base + j * CHUNK, CHUNK)], idx_v, sem)

        def finish(j, idx_v, sem):
            pltpu.make_async_copy(
                dst_hbm.at[pl.ds(base + j * CHUNK, CHUNK)], idx_v, sem
            ).wait()
            pltpu.sync_copy(ones_v, acc_sh.at[idx_v], add=True)

        start(0, idx_a, sa)

        def body(t, _):
            j0 = 2 * t
            start(j0 + 1, idx_b, sb)
            finish(j0, idx_a, sa)

            @pl.when(t + 1 < CPT // 2)
            def _():
                start(j0 + 2, idx_a, sa)

            finish(j0 + 1, idx_b, sb)
            return 0

        lax.fori_loop(0, CPT // 2, body, 0)
        plsc.subcore_barrier()
        pltpu.sync_copy(
            acc_sh.at[pl.ds(s * RPT, RPT)],
            out_hbm.at[pl.ds(c * NPAD + s * RPT, RPT)],
        )

    return k(dst1d)


def _msg_partials(hp, src1d, dst1d, feat):
    params = (
        pltpu.CompilerParams(use_tc_tiling_on_sc=False) if feat != 128 else None
    )

    nb = 2 if feat == 128 else 4

    @functools.partial(
        pl.kernel,
        out_type=jax.ShapeDtypeStruct((NC, NPAD, feat), jnp.float32),
        mesh=_mesh(),
        compiler_params=params,
        scratch_types=[
            pltpu.VMEM((CPT * CHUNK,), jnp.int32),
            [pltpu.VMEM((CHUNK,), jnp.int32)] * nb,
            [pltpu.VMEM((CHUNK, feat), jnp.float32)] * nb,
            pltpu.VMEM_SHARED((NPAD, feat), jnp.float32),
            [pltpu.SemaphoreType.DMA] * nb,
            [pltpu.SemaphoreType.DMA] * nb,
        ],
    )
    def k(h_hbm, src_hbm, dst_hbm, out_hbm, si_all, di_v, rows_v, acc_sh,
          sg, sd):
        c = lax.axis_index("c")
        s = lax.axis_index("s")

        def zrow(i, _):
            def zcol(t, _):
                rows_v[0][i, pl.ds(t * 16, 16)] = jnp.zeros((16,), jnp.float32)
                return 0

            lax.fori_loop(0, feat // 16, zcol, 0)
            return 0

        lax.fori_loop(0, CHUNK, zrow, 0)

        def zacc(q, _):
            pltpu.sync_copy(rows_v[0], acc_sh.at[pl.ds(s * RPT + q * CHUNK, CHUNK)])
            return 0

        lax.fori_loop(0, RPT // CHUNK, zacc, 0)
        plsc.subcore_barrier()

        base = (c * NS + s) * CPT * CHUNK
        pltpu.sync_copy(src_hbm.at[pl.ds(base, CPT * CHUNK)], si_all)

        def start(j, b):
            pltpu.async_copy(
                dst_hbm.at[pl.ds(base + j * CHUNK, CHUNK)], di_v[b], sd[b]
            )
            pltpu.async_copy(
                h_hbm.at[si_all.at[pl.ds(j * CHUNK, CHUNK)]], rows_v[b], sg[b]
            )

        def finish(j, b):
            pltpu.make_async_copy(
                dst_hbm.at[pl.ds(base + j * CHUNK, CHUNK)], di_v[b], sd[b]
            ).wait()
            pltpu.make_async_copy(
                h_hbm.at[si_all.at[pl.ds(j * CHUNK, CHUNK)]], rows_v[b], sg[b]
            ).wait()
            pltpu.sync_copy(rows_v[b], acc_sh.at[di_v[b]], add=True)

        for i in range(nb - 1):
            start(i, i)

        def body(t, _):
            j0 = nb * t
            for i in range(nb):
                j = j0 + i
                jn = j + nb - 1

                @pl.when(jn < CPT)
                def _():
                    start(jn, (i + nb - 1) % nb)

                finish(j, i)
            return 0

        lax.fori_loop(0, CPT // nb, body, 0)
        plsc.subcore_barrier()
        pltpu.sync_copy(
            acc_sh.at[pl.ds(s * RPT, RPT)], out_hbm.at[c, pl.ds(s * RPT, RPT)]
        )

    return k(hp, src1d, dst1d)


def _tc_first(dp, xp, W1):
    def body(dp_ref, x_ref, w_ref, o_ref):
        deg = dp_ref[0, :] + dp_ref[1, :] + 1.0
        dinv = lax.rsqrt(deg)
        h = jnp.dot(x_ref[...], w_ref[...], preferred_element_type=jnp.float32)
        o_ref[...] = h * dinv[:, None]

    return pl.pallas_call(
        body, out_shape=jax.ShapeDtypeStruct((NPAD, 128), jnp.float32)
    )(dp, xp, W1)


def _tc_mid(P, hp, dp, b1, W2):
    def body(p_ref, h_ref, dp_ref, b_ref, w_ref, o_ref):
        deg = dp_ref[0, :] + dp_ref[1, :] + 1.0
        dinv = lax.rsqrt(deg)
        z = (p_ref[0] + p_ref[1] + h_ref[...]) * dinv[:, None] + b_ref[...][None, :]
        z = jnp.maximum(z, 0.0)
        h2 = jnp.dot(z, w_ref[...], preferred_element_type=jnp.float32)
        o_ref[...] = h2 * dinv[:, None]

    return pl.pallas_call(
        body, out_shape=jax.ShapeDtypeStruct((NPAD, 64), jnp.float32)
    )(P, hp, dp, b1, W2)


def _tc_last(P, hp, dp, b2):
    def body(p_ref, h_ref, dp_ref, b_ref, o_ref):
        deg = dp_ref[0, :] + dp_ref[1, :] + 1.0
        dinv = lax.rsqrt(deg)
        z = (p_ref[0] + p_ref[1] + h_ref[...]) * dinv[:, None] + b_ref[...][None, :]
        m = jnp.max(z, axis=1, keepdims=True)
        e = jnp.exp(z - m)
        lse = jnp.log(jnp.sum(e, axis=1, keepdims=True))
        o_ref[...] = z - m - lse

    return pl.pallas_call(
        body, out_shape=jax.ShapeDtypeStruct((NPAD, 64), jnp.float32)
    )(P, hp, dp, b2)


@jax.jit
def _run(x, edge_index, W1, b1, W2, b2):
    src = edge_index[0]
    dst = edge_index[1]
    pad = N + (jnp.arange(EPAD - src.shape[0], dtype=jnp.int32) % (NPAD - N))
    src1d = jnp.concatenate([src, pad])
    dst1d = jnp.concatenate([dst, pad])
    xp = jnp.zeros((NPAD, x.shape[1]), x.dtype).at[:N].set(x)

    dp = _deg_partials(dst1d).reshape(NC, NPAD)
    hp1 = _tc_first(dp, xp, W1)
    P1 = _msg_partials(hp1, src1d, dst1d, 128)
    hp2 = _tc_mid(P1, hp1, dp, b1, W2)
    P2 = _msg_partials(hp2, src1d, dst1d, 64)
    out = _tc_last(P2, hp2, dp, b2)
    return out[:N]


def kernel(x, edge_index, W1, b1, W2, b2):
    return _run(x, edge_index, W1, b1, W2, b2)


# speedup vs baseline: 40.7655x; 1.0444x over previous
import functools

import jax
import jax.numpy as jnp
from jax import lax
from jax.experimental import pallas as pl
from jax.experimental.pallas import tpu as pltpu
from jax.experimental.pallas import tpu_sc as plsc

N = 10000
NC, NS = 2, 16
NT = NC * NS
NPAD = 10240
CHUNK = 128
CPT = 80
EPAD = NT * CPT * CHUNK
RPT = NPAD // NS


def _mesh():
    return plsc.VectorSubcoreMesh(
        core_axis_name="c", subcore_axis_name="s", num_cores=NC, num_subcores=NS
    )


def _deg_partials(dst1d):
    @functools.partial(
        pl.kernel,
        out_type=jax.ShapeDtypeStruct((NC * NPAD,), jnp.float32),
        mesh=_mesh(),
        scratch_types=[
            [pltpu.VMEM((CHUNK,), jnp.int32)] * 4,
            pltpu.VMEM((CHUNK,), jnp.float32),
            pltpu.VMEM((RPT,), jnp.float32),
            pltpu.VMEM_SHARED((NPAD,), jnp.float32),
            [pltpu.SemaphoreType.DMA] * 4,
        ],
    )
    def k(dst_hbm, out_hbm, idx_v, ones_v, z_v, acc_sh, sd):
        c = lax.axis_index("c")
        s = lax.axis_index("s")

        def fill_ones(i, _):
            ones_v[pl.ds(i * 16, 16)] = jnp.ones((16,), jnp.float32)
            return 0

        lax.fori_loop(0, CHUNK // 16, fill_ones, 0)

        def fill_z(i, _):
            z_v[pl.ds(i * 16, 16)] = jnp.zeros((16,), jnp.float32)
            return 0

        lax.fori_loop(0, RPT // 16, fill_z, 0)
        pltpu.sync_copy(z_v, acc_sh.at[pl.ds(s * RPT, RPT)])
        plsc.subcore_barrier()

        base = (c * NS + s) * CPT * CHUNK
        nb = 4

        def start(j, b):
            pltpu.async_copy(
                dst_hbm.at[pl.ds(base + j * CHUNK, CHUNK)], idx_v[b], sd[b]
            )

        def finish(j, b):
            pltpu.make_async_copy(
                dst_hbm.at[pl.ds(base + j * CHUNK, CHUNK)], idx_v[b], sd[b]
            ).wait()
            pltpu.sync_copy(ones_v, acc_sh.at[idx_v[b]], add=True)

        for i in range(nb - 1):
            start(i, i)

        def body(t, _):
            j0 = nb * t
            for i in range(nb):
                j = j0 + i
                jn = j + nb - 1

                @pl.when(jn < CPT)
                def _():
                    start(jn, (i + nb - 1) % nb)

                finish(j, i)
            return 0

        lax.fori_loop(0, CPT // nb, body, 0)
        plsc.subcore_barrier()
        pltpu.sync_copy(
            acc_sh.at[pl.ds(s * RPT, RPT)],
            out_hbm.at[pl.ds(c * NPAD + s * RPT, RPT)],
        )

    return k(dst1d)


def _msg_partials(hp, src1d, dst1d, feat):
    params = (
        pltpu.CompilerParams(use_tc_tiling_on_sc=False) if feat != 128 else None
    )

    nb = 2 if feat == 128 else 4

    @functools.partial(
        pl.kernel,
        out_type=jax.ShapeDtypeStruct((NC, NPAD, feat), jnp.float32),
        mesh=_mesh(),
        compiler_params=params,
        scratch_types=[
            pltpu.VMEM((CPT * CHUNK,), jnp.int32),
            [pltpu.VMEM((CHUNK,), jnp.int32)] * nb,
            [pltpu.VMEM((CHUNK, feat), jnp.float32)] * nb,
            pltpu.VMEM_SHARED((NPAD, feat), jnp.float32),
            [pltpu.SemaphoreType.DMA] * nb,
            [pltpu.SemaphoreType.DMA] * nb,
        ],
    )
    def k(h_hbm, src_hbm, dst_hbm, out_hbm, si_all, di_v, rows_v, acc_sh,
          sg, sd):
        c = lax.axis_index("c")
        s = lax.axis_index("s")

        def zrow(i, _):
            def zcol(t, _):
                rows_v[0][i, pl.ds(t * 16, 16)] = jnp.zeros((16,), jnp.float32)
                return 0

            lax.fori_loop(0, feat // 16, zcol, 0)
            return 0

        lax.fori_loop(0, CHUNK, zrow, 0)

        def zacc(q, _):
            pltpu.sync_copy(rows_v[0], acc_sh.at[pl.ds(s * RPT + q * CHUNK, CHUNK)])
            return 0

        lax.fori_loop(0, RPT // CHUNK, zacc, 0)
        plsc.subcore_barrier()

        base = (c * NS + s) * CPT * CHUNK
        pltpu.sync_copy(src_hbm.at[pl.ds(base, CPT * CHUNK)], si_all)

        def start(j, b):
            pltpu.async_copy(
                dst_hbm.at[pl.ds(base + j * CHUNK, CHUNK)], di_v[b], sd[b]
            )
            pltpu.async_copy(
                h_hbm.at[si_all.at[pl.ds(j * CHUNK, CHUNK)]], rows_v[b], sg[b]
            )

        def finish(j, b):
            pltpu.make_async_copy(
                dst_hbm.at[pl.ds(base + j * CHUNK, CHUNK)], di_v[b], sd[b]
            ).wait()
            pltpu.make_async_copy(
                h_hbm.at[si_all.at[pl.ds(j * CHUNK, CHUNK)]], rows_v[b], sg[b]
            ).wait()
            pltpu.sync_copy(rows_v[b], acc_sh.at[di_v[b]], add=True)

        for i in range(nb - 1):
            start(i, i)

        def body(t, _):
            j0 = nb * t
            for i in range(nb):
                j = j0 + i
                jn = j + nb - 1

                @pl.when(jn < CPT)
                def _():
                    start(jn, (i + nb - 1) % nb)

                finish(j, i)
            return 0

        lax.fori_loop(0, CPT // nb, body, 0)
        plsc.subcore_barrier()
        pltpu.sync_copy(
            acc_sh.at[pl.ds(s * RPT, RPT)], out_hbm.at[c, pl.ds(s * RPT, RPT)]
        )

    return k(hp, src1d, dst1d)


def _tc_first(dp, xp, W1):
    def body(dp_ref, x_ref, w_ref, o_ref):
        deg = dp_ref[0, :] + dp_ref[1, :] + 1.0
        dinv = lax.rsqrt(deg)
        h = jnp.dot(x_ref[...], w_ref[...], preferred_element_type=jnp.float32)
        o_ref[...] = h * dinv[:, None]

    return pl.pallas_call(
        body, out_shape=jax.ShapeDtypeStruct((NPAD, 128), jnp.float32)
    )(dp, xp, W1)


def _tc_mid(P, hp, dp, b1, W2):
    def body(p_ref, h_ref, dp_ref, b_ref, w_ref, o_ref):
        deg = dp_ref[0, :] + dp_ref[1, :] + 1.0
        dinv = lax.rsqrt(deg)
        z = (p_ref[0] + p_ref[1] + h_ref[...]) * dinv[:, None] + b_ref[...][None, :]
        z = jnp.maximum(z, 0.0)
        h2 = jnp.dot(z, w_ref[...], preferred_element_type=jnp.float32)
        o_ref[...] = h2 * dinv[:, None]

    return pl.pallas_call(
        body, out_shape=jax.ShapeDtypeStruct((NPAD, 64), jnp.float32)
    )(P, hp, dp, b1, W2)


def _tc_last(P, hp, dp, b2):
    def body(p_ref, h_ref, dp_ref, b_ref, o_ref):
        deg = dp_ref[0, :] + dp_ref[1, :] + 1.0
        dinv = lax.rsqrt(deg)
        z = (p_ref[0] + p_ref[1] + h_ref[...]) * dinv[:, None] + b_ref[...][None, :]
        m = jnp.max(z, axis=1, keepdims=True)
        e = jnp.exp(z - m)
        lse = jnp.log(jnp.sum(e, axis=1, keepdims=True))
        o_ref[...] = z - m - lse

    return pl.pallas_call(
        body, out_shape=jax.ShapeDtypeStruct((NPAD, 64), jnp.float32)
    )(P, hp, dp, b2)


@jax.jit
def _run(x, edge_index, W1, b1, W2, b2):
    src = edge_index[0]
    dst = edge_index[1]
    pad = N + (jnp.arange(EPAD - src.shape[0], dtype=jnp.int32) % (NPAD - N))
    src1d = jnp.concatenate([src, pad])
    dst1d = jnp.concatenate([dst, pad])
    xp = jnp.zeros((NPAD, x.shape[1]), x.dtype).at[:N].set(x)

    dp = _deg_partials(dst1d).reshape(NC, NPAD)
    hp1 = _tc_first(dp, xp, W1)
    P1 = _msg_partials(hp1, src1d, dst1d, 128)
    hp2 = _tc_mid(P1, hp1, dp, b1, W2)
    P2 = _msg_partials(hp2, src1d, dst1d, 64)
    out = _tc_last(P2, hp2, dp, b2)
    return out[:N]


def kernel(x, edge_index, W1, b1, W2, b2):
    return _run(x, edge_index, W1, b1, W2, b2)


# speedup vs baseline: 41.0084x; 1.0060x over previous
import functools

import jax
import jax.numpy as jnp
from jax import lax
from jax.experimental import pallas as pl
from jax.experimental.pallas import tpu as pltpu
from jax.experimental.pallas import tpu_sc as plsc

N = 10000
NC, NS = 2, 16
NT = NC * NS
NPAD = 10240
CHUNK = 64
CPT = 160
EPAD = NT * CPT * CHUNK
RPT = NPAD // NS


def _mesh():
    return plsc.VectorSubcoreMesh(
        core_axis_name="c", subcore_axis_name="s", num_cores=NC, num_subcores=NS
    )


def _deg_partials(dst1d):
    @functools.partial(
        pl.kernel,
        out_type=jax.ShapeDtypeStruct((NC * NPAD,), jnp.float32),
        mesh=_mesh(),
        scratch_types=[
            [pltpu.VMEM((CHUNK,), jnp.int32)] * 4,
            pltpu.VMEM((CHUNK,), jnp.float32),
            pltpu.VMEM((RPT,), jnp.float32),
            pltpu.VMEM_SHARED((NPAD,), jnp.float32),
            [pltpu.SemaphoreType.DMA] * 4,
        ],
    )
    def k(dst_hbm, out_hbm, idx_v, ones_v, z_v, acc_sh, sd):
        c = lax.axis_index("c")
        s = lax.axis_index("s")

        def fill_ones(i, _):
            ones_v[pl.ds(i * 16, 16)] = jnp.ones((16,), jnp.float32)
            return 0

        lax.fori_loop(0, CHUNK // 16, fill_ones, 0)

        def fill_z(i, _):
            z_v[pl.ds(i * 16, 16)] = jnp.zeros((16,), jnp.float32)
            return 0

        lax.fori_loop(0, RPT // 16, fill_z, 0)
        pltpu.sync_copy(z_v, acc_sh.at[pl.ds(s * RPT, RPT)])
        plsc.subcore_barrier()

        base = (c * NS + s) * CPT * CHUNK
        nb = 4

        def start(j, b):
            pltpu.async_copy(
                dst_hbm.at[pl.ds(base + j * CHUNK, CHUNK)], idx_v[b], sd[b]
            )

        def finish(j, b):
            pltpu.make_async_copy(
                dst_hbm.at[pl.ds(base + j * CHUNK, CHUNK)], idx_v[b], sd[b]
            ).wait()
            pltpu.sync_copy(ones_v, acc_sh.at[idx_v[b]], add=True)

        for i in range(nb - 1):
            start(i, i)

        def body(t, _):
            j0 = nb * t
            for i in range(nb):
                j = j0 + i
                jn = j + nb - 1

                @pl.when(jn < CPT)
                def _():
                    start(jn, (i + nb - 1) % nb)

                finish(j, i)
            return 0

        lax.fori_loop(0, CPT // nb, body, 0)
        plsc.subcore_barrier()
        pltpu.sync_copy(
            acc_sh.at[pl.ds(s * RPT, RPT)],
            out_hbm.at[pl.ds(c * NPAD + s * RPT, RPT)],
        )

    return k(dst1d)


def _msg_partials(hp, src1d, dst1d, feat):
    params = (
        pltpu.CompilerParams(use_tc_tiling_on_sc=False) if feat != 128 else None
    )

    nb = 4 if feat == 128 else 8

    @functools.partial(
        pl.kernel,
        out_type=jax.ShapeDtypeStruct((NC, NPAD, feat), jnp.float32),
        mesh=_mesh(),
        compiler_params=params,
        scratch_types=[
            pltpu.VMEM((CPT * CHUNK,), jnp.int32),
            [pltpu.VMEM((CHUNK,), jnp.int32)] * nb,
            [pltpu.VMEM((CHUNK, feat), jnp.float32)] * nb,
            pltpu.VMEM_SHARED((NPAD, feat), jnp.float32),
            [pltpu.SemaphoreType.DMA] * nb,
            [pltpu.SemaphoreType.DMA] * nb,
        ],
    )
    def k(h_hbm, src_hbm, dst_hbm, out_hbm, si_all, di_v, rows_v, acc_sh,
          sg, sd):
        c = lax.axis_index("c")
        s = lax.axis_index("s")

        def zrow(i, _):
            def zcol(t, _):
                rows_v[0][i, pl.ds(t * 16, 16)] = jnp.zeros((16,), jnp.float32)
                return 0

            lax.fori_loop(0, feat // 16, zcol, 0)
            return 0

        lax.fori_loop(0, CHUNK, zrow, 0)

        def zacc(q, _):
            pltpu.sync_copy(rows_v[0], acc_sh.at[pl.ds(s * RPT + q * CHUNK, CHUNK)])
            return 0

        lax.fori_loop(0, RPT // CHUNK, zacc, 0)
        plsc.subcore_barrier()

        base = (c * NS + s) * CPT * CHUNK
        pltpu.sync_copy(src_hbm.at[pl.ds(base, CPT * CHUNK)], si_all)

        def start(j, b):
            pltpu.async_copy(
                dst_hbm.at[pl.ds(base + j * CHUNK, CHUNK)], di_v[b], sd[b]
            )
            pltpu.async_copy(
                h_hbm.at[si_all.at[pl.ds(j * CHUNK, CHUNK)]], rows_v[b], sg[b]
            )

        def finish(j, b):
            pltpu.make_async_copy(
                dst_hbm.at[pl.ds(base + j * CHUNK, CHUNK)], di_v[b], sd[b]
            ).wait()
            pltpu.make_async_copy(
                h_hbm.at[si_all.at[pl.ds(j * CHUNK, CHUNK)]], rows_v[b], sg[b]
            ).wait()
            pltpu.sync_copy(rows_v[b], acc_sh.at[di_v[b]], add=True)

        for i in range(nb - 1):
            start(i, i)

        def body(t, _):
            j0 = nb * t
            for i in range(nb):
                j = j0 + i
                jn = j + nb - 1

                @pl.when(jn < CPT)
                def _():
                    start(jn, (i + nb - 1) % nb)

                finish(j, i)
            return 0

        lax.fori_loop(0, CPT // nb, body, 0)
        plsc.subcore_barrier()
        pltpu.sync_copy(
            acc_sh.at[pl.ds(s * RPT, RPT)], out_hbm.at[c, pl.ds(s * RPT, RPT)]
        )

    return k(hp, src1d, dst1d)


def _tc_first(dp, xp, W1):
    def body(dp_ref, x_ref, w_ref, o_ref):
        deg = dp_ref[0, :] + dp_ref[1, :] + 1.0
        dinv = lax.rsqrt(deg)
        h = jnp.dot(x_ref[...], w_ref[...], preferred_element_type=jnp.float32)
        o_ref[...] = h * dinv[:, None]

    return pl.pallas_call(
        body, out_shape=jax.ShapeDtypeStruct((NPAD, 128), jnp.float32)
    )(dp, xp, W1)


def _tc_mid(P, hp, dp, b1, W2):
    def body(p_ref, h_ref, dp_ref, b_ref, w_ref, o_ref):
        deg = dp_ref[0, :] + dp_ref[1, :] + 1.0
        dinv = lax.rsqrt(deg)
        z = (p_ref[0] + p_ref[1] + h_ref[...]) * dinv[:, None] + b_ref[...][None, :]
        z = jnp.maximum(z, 0.0)
        h2 = jnp.dot(z, w_ref[...], preferred_element_type=jnp.float32)
        o_ref[...] = h2 * dinv[:, None]

    return pl.pallas_call(
        body, out_shape=jax.ShapeDtypeStruct((NPAD, 64), jnp.float32)
    )(P, hp, dp, b1, W2)


def _tc_last(P, hp, dp, b2):
    def body(p_ref, h_ref, dp_ref, b_ref, o_ref):
        deg = dp_ref[0, :] + dp_ref[1, :] + 1.0
        dinv = lax.rsqrt(deg)
        z = (p_ref[0] + p_ref[1] + h_ref[...]) * dinv[:, None] + b_ref[...][None, :]
        m = jnp.max(z, axis=1, keepdims=True)
        e = jnp.exp(z - m)
        lse = jnp.log(jnp.sum(e, axis=1, keepdims=True))
        o_ref[...] = z - m - lse

    return pl.pallas_call(
        body, out_shape=jax.ShapeDtypeStruct((NPAD, 64), jnp.float32)
    )(P, hp, dp, b2)


@jax.jit
def _run(x, edge_index, W1, b1, W2, b2):
    src = edge_index[0]
    dst = edge_index[1]
    pad = N + (jnp.arange(EPAD - src.shape[0], dtype=jnp.int32) % (NPAD - N))
    src1d = jnp.concatenate([src, pad])
    dst1d = jnp.concatenate([dst, pad])
    xp = jnp.zeros((NPAD, x.shape[1]), x.dtype).at[:N].set(x)

    dp = _deg_partials(dst1d).reshape(NC, NPAD)
    hp1 = _tc_first(dp, xp, W1)
    P1 = _msg_partials(hp1, src1d, dst1d, 128)
    hp2 = _tc_mid(P1, hp1, dp, b1, W2)
    P2 = _msg_partials(hp2, src1d, dst1d, 64)
    out = _tc_last(P2, hp2, dp, b2)
    return out[:N]


def kernel(x, edge_index, W1, b1, W2, b2):
    return _run(x, edge_index, W1, b1, W2, b2)


# speedup vs baseline: 41.2298x; 1.0054x over previous
import functools

import jax
import jax.numpy as jnp
from jax import lax
from jax.experimental import pallas as pl
from jax.experimental.pallas import tpu as pltpu
from jax.experimental.pallas import tpu_sc as plsc

N = 10000
NC, NS = 2, 16
NT = NC * NS
NPAD = 10240
CHUNK = 64
CPT = 160
EPAD = NT * CPT * CHUNK
RPT = NPAD // NS


def _mesh():
    return plsc.VectorSubcoreMesh(
        core_axis_name="c", subcore_axis_name="s", num_cores=NC, num_subcores=NS
    )


def _deg_partials(dst1d):
    @functools.partial(
        pl.kernel,
        out_type=jax.ShapeDtypeStruct((NC * NPAD,), jnp.float32),
        mesh=_mesh(),
        scratch_types=[
            [pltpu.VMEM((CHUNK,), jnp.int32)] * 4,
            pltpu.VMEM((CHUNK,), jnp.float32),
            pltpu.VMEM((RPT,), jnp.float32),
            pltpu.VMEM_SHARED((NPAD,), jnp.float32),
            [pltpu.SemaphoreType.DMA] * 4,
        ],
    )
    def k(dst_hbm, out_hbm, idx_v, ones_v, z_v, acc_sh, sd):
        c = lax.axis_index("c")
        s = lax.axis_index("s")

        def fill_ones(i, _):
            ones_v[pl.ds(i * 16, 16)] = jnp.ones((16,), jnp.float32)
            return 0

        lax.fori_loop(0, CHUNK // 16, fill_ones, 0)

        def fill_z(i, _):
            z_v[pl.ds(i * 16, 16)] = jnp.zeros((16,), jnp.float32)
            return 0

        lax.fori_loop(0, RPT // 16, fill_z, 0)
        pltpu.sync_copy(z_v, acc_sh.at[pl.ds(s * RPT, RPT)])
        plsc.subcore_barrier()

        base = (c * NS + s) * CPT * CHUNK
        nb = 4

        def start(j, b):
            pltpu.async_copy(
                dst_hbm.at[pl.ds(base + j * CHUNK, CHUNK)], idx_v[b], sd[b]
            )

        def finish(j, b):
            pltpu.make_async_copy(
                dst_hbm.at[pl.ds(base + j * CHUNK, CHUNK)], idx_v[b], sd[b]
            ).wait()
            pltpu.sync_copy(ones_v, acc_sh.at[idx_v[b]], add=True)

        for i in range(nb - 1):
            start(i, i)

        def body(t, _):
            j0 = nb * t
            for i in range(nb):
                j = j0 + i
                jn = j + nb - 1

                @pl.when(jn < CPT)
                def _():
                    start(jn, (i + nb - 1) % nb)

                finish(j, i)
            return 0

        lax.fori_loop(0, CPT // nb, body, 0)
        plsc.subcore_barrier()
        pltpu.sync_copy(
            acc_sh.at[pl.ds(s * RPT, RPT)],
            out_hbm.at[pl.ds(c * NPAD + s * RPT, RPT)],
        )

    return k(dst1d)


def _msg_partials(hp, src1d, dst1d, feat):
    params = (
        pltpu.CompilerParams(use_tc_tiling_on_sc=False) if feat != 128 else None
    )

    nb = 4 if feat == 128 else 8

    @functools.partial(
        pl.kernel,
        out_type=jax.ShapeDtypeStruct((NC, NPAD, feat), jnp.float32),
        mesh=_mesh(),
        compiler_params=params,
        scratch_types=[
            pltpu.VMEM((CPT * CHUNK,), jnp.int32),
            [pltpu.VMEM((CHUNK,), jnp.int32)] * nb,
            [pltpu.VMEM((CHUNK, feat), jnp.float32)] * nb,
            pltpu.VMEM_SHARED((NPAD, feat), jnp.float32),
            [pltpu.SemaphoreType.DMA] * nb,
            [pltpu.SemaphoreType.DMA] * nb,
        ],
    )
    def k(h_hbm, src_hbm, dst_hbm, out_hbm, si_all, di_v, rows_v, acc_sh,
          sg, sd):
        c = lax.axis_index("c")
        s = lax.axis_index("s")

        def zrow(i, _):
            def zcol(t, _):
                rows_v[0][i, pl.ds(t * 16, 16)] = jnp.zeros((16,), jnp.float32)
                return 0

            lax.fori_loop(0, feat // 16, zcol, 0)
            return 0

        lax.fori_loop(0, CHUNK, zrow, 0)

        def zacc(q, _):
            pltpu.sync_copy(rows_v[0], acc_sh.at[pl.ds(s * RPT + q * CHUNK, CHUNK)])
            return 0

        lax.fori_loop(0, RPT // CHUNK, zacc, 0)
        plsc.subcore_barrier()

        base = (c * NS + s) * CPT * CHUNK
        pltpu.sync_copy(src_hbm.at[pl.ds(base, CPT * CHUNK)], si_all)

        def start(j, b):
            pltpu.async_copy(
                dst_hbm.at[pl.ds(base + j * CHUNK, CHUNK)], di_v[b], sd[b]
            )
            pltpu.async_copy(
                h_hbm.at[si_all.at[pl.ds(j * CHUNK, CHUNK)]], rows_v[b], sg[b]
            )

        def finish(j, b):
            pltpu.make_async_copy(
                dst_hbm.at[pl.ds(base + j * CHUNK, CHUNK)], di_v[b], sd[b]
            ).wait()
            pltpu.make_async_copy(
                h_hbm.at[si_all.at[pl.ds(j * CHUNK, CHUNK)]], rows_v[b], sg[b]
            ).wait()
            pltpu.sync_copy(rows_v[b], acc_sh.at[di_v[b]], add=True)

        for i in range(nb - 1):
            start(i, i)

        def body(t, _):
            j0 = nb * t
            for i in range(nb):
                j = j0 + i
                jn = j + nb - 1

                @pl.when(jn < CPT)
                def _():
                    start(jn, (i + nb - 1) % nb)

                finish(j, i)
            return 0

        lax.fori_loop(0, CPT // nb, body, 0)
        plsc.subcore_barrier()
        pltpu.sync_copy(
            acc_sh.at[pl.ds(s * RPT, RPT)], out_hbm.at[c, pl.ds(s * RPT, RPT)]
        )

    return k(hp, src1d, dst1d)


def _tc_first(dp, x, W1):
    def body(dp_ref, x_ref, w_ref, o_ref):
        deg = dp_ref[0, :N] + dp_ref[1, :N] + 1.0
        dinv = lax.rsqrt(deg)
        h = jnp.dot(x_ref[...], w_ref[...], preferred_element_type=jnp.float32)
        hp = h * dinv[:, None]
        o_ref[...] = jnp.concatenate(
            [hp, jnp.zeros((NPAD - N, 128), jnp.float32)], axis=0
        )

    return pl.pallas_call(
        body, out_shape=jax.ShapeDtypeStruct((NPAD, 128), jnp.float32)
    )(dp, x, W1)


def _tc_mid(P, hp, dp, b1, W2):
    def body(p_ref, h_ref, dp_ref, b_ref, w_ref, o_ref):
        deg = dp_ref[0, :] + dp_ref[1, :] + 1.0
        dinv = lax.rsqrt(deg)
        z = (p_ref[0] + p_ref[1] + h_ref[...]) * dinv[:, None] + b_ref[...][None, :]
        z = jnp.maximum(z, 0.0)
        h2 = jnp.dot(z, w_ref[...], preferred_element_type=jnp.float32)
        o_ref[...] = h2 * dinv[:, None]

    return pl.pallas_call(
        body, out_shape=jax.ShapeDtypeStruct((NPAD, 64), jnp.float32)
    )(P, hp, dp, b1, W2)


def _tc_last(P, hp, dp, b2):
    def body(p_ref, h_ref, dp_ref, b_ref, o_ref):
        deg = dp_ref[0, :N] + dp_ref[1, :N] + 1.0
        dinv = lax.rsqrt(deg)
        z = (p_ref[0, :N] + p_ref[1, :N] + h_ref[:N]) * dinv[:, None]
        z = z + b_ref[...][None, :]
        m = jnp.max(z, axis=1, keepdims=True)
        e = jnp.exp(z - m)
        lse = jnp.log(jnp.sum(e, axis=1, keepdims=True))
        o_ref[...] = z - m - lse

    return pl.pallas_call(
        body, out_shape=jax.ShapeDtypeStruct((N, 64), jnp.float32)
    )(P, hp, dp, b2)


@jax.jit
def _run(x, edge_index, W1, b1, W2, b2):
    src = edge_index[0]
    dst = edge_index[1]
    pad = N + (jnp.arange(EPAD - src.shape[0], dtype=jnp.int32) % (NPAD - N))
    src1d = jnp.concatenate([src, pad])
    dst1d = jnp.concatenate([dst, pad])

    dp = _deg_partials(dst1d).reshape(NC, NPAD)
    hp1 = _tc_first(dp, x, W1)
    P1 = _msg_partials(hp1, src1d, dst1d, 128)
    hp2 = _tc_mid(P1, hp1, dp, b1, W2)
    P2 = _msg_partials(hp2, src1d, dst1d, 64)
    return _tc_last(P2, hp2, dp, b2)


def kernel(x, edge_index, W1, b1, W2, b2):
    return _run(x, edge_index, W1, b1, W2, b2)
